# Initial kernel scaffold; baseline (speedup 1.0000x reference)
import functools

import jax
import jax.numpy as jnp
from jax import lax
from jax.experimental import pallas as pl
from jax.experimental.pallas import tpu as pltpu
from jax.experimental.pallas import tpu_sc as plsc

N = 100000
E = 3200000
K = 5

NUM_CORES = 2
NUM_SUBCORES = 16
NUM_TILES = NUM_CORES * NUM_SUBCORES

NODE_SLICE = 6256
NP = NUM_SUBCORES * NODE_SLICE
NPR = NP // 128

CHUNK = 2048
TILE_CHUNKS = 49
TILE_EDGES = CHUNK * TILE_CHUNKS
EP = NUM_TILES * TILE_EDGES
EPR = EP // 128


def _sc_mesh():
    return plsc.VectorSubcoreMesh(
        core_axis_name="c", subcore_axis_name="s",
        num_cores=NUM_CORES, num_subcores=NUM_SUBCORES)


def _zero_spmem_slice(zbuf, agg_sh, tid):
    def zb(i, _):
        zbuf[pl.ds(i * 16, 16)] = jnp.zeros((16,), jnp.float32)
        return 0
    lax.fori_loop(0, NODE_SLICE // 16, zb, 0)
    pltpu.sync_copy(zbuf, agg_sh.at[pl.ds(tid * NODE_SLICE, NODE_SLICE)])


def _sc_deg_body(col_hbm, w_hbm, out_hbm, col_v, w_v, zbuf, agg_sh):
    cid = lax.axis_index("c")
    tid = lax.axis_index("s")
    wid = cid * NUM_SUBCORES + tid
    rbase = wid * (TILE_EDGES // 128)
    _zero_spmem_slice(zbuf, agg_sh, tid)
    plsc.subcore_barrier()

    def chunk_body(c, _):
        rb = rbase + c * (CHUNK // 128)
        pltpu.sync_copy(col_hbm.at[pl.ds(rb, CHUNK // 128), :], col_v)
        pltpu.sync_copy(w_hbm.at[pl.ds(rb, CHUNK // 128), :], w_v)
        pltpu.sync_copy(w_v, agg_sh.at[col_v], add=True)
        return 0
    lax.fori_loop(0, TILE_CHUNKS, chunk_body, 0)

    plsc.subcore_barrier()
    nb = tid * NODE_SLICE
    pltpu.sync_copy(agg_sh.at[pl.ds(nb, NODE_SLICE)],
                    out_hbm.at[cid, pl.ds(nb, NODE_SLICE)])


def _sc_prop_body(row_hbm, col_hbm, w_hbm, y_hbm, out_hbm,
                  y_v, row_v, col_v, w_v, msg_v, zbuf, agg_sh):
    cid = lax.axis_index("c")
    tid = lax.axis_index("s")
    wid = cid * NUM_SUBCORES + tid
    ebase = wid * TILE_EDGES
    rbase = wid * (TILE_EDGES // 128)
    _zero_spmem_slice(zbuf, agg_sh, tid)
    pltpu.sync_copy(y_hbm, y_v)
    plsc.subcore_barrier()

    def chunk_body(c, _):
        eb = ebase + c * CHUNK
        rb = rbase + c * (CHUNK // 128)
        pltpu.sync_copy(row_hbm.at[pl.ds(eb, CHUNK)], row_v)
        pltpu.sync_copy(col_hbm.at[pl.ds(rb, CHUNK // 128), :], col_v)
        pltpu.sync_copy(w_hbm.at[pl.ds(rb, CHUNK // 128), :], w_v)

        def g(i, _):
            r = i // 8
            lc = (i % 8) * 16
            idx = row_v[pl.ds(i * 16, 16)]
            xv = plsc.load_gather(y_v, [idx])
            msg_v[r, pl.ds(lc, 16)] = xv * w_v[r, pl.ds(lc, 16)]
            return 0
        lax.fori_loop(0, CHUNK // 16, g, 0)
        pltpu.sync_copy(msg_v, agg_sh.at[col_v], add=True)
        return 0
    lax.fori_loop(0, TILE_CHUNKS, chunk_body, 0)

    plsc.subcore_barrier()
    nb = tid * NODE_SLICE
    pltpu.sync_copy(agg_sh.at[pl.ds(nb, NODE_SLICE)],
                    out_hbm.at[cid, pl.ds(nb, NODE_SLICE)])


def _sc_deg(col2d, w2d):
    f = pl.kernel(
        _sc_deg_body,
        out_type=jax.ShapeDtypeStruct((NUM_CORES, NP), jnp.float32),
        mesh=_sc_mesh(),
        scratch_types=[
            pltpu.VMEM((CHUNK // 128, 128), jnp.int32),
            pltpu.VMEM((CHUNK // 128, 128), jnp.float32),
            pltpu.VMEM((NODE_SLICE,), jnp.float32),
            pltpu.VMEM_SHARED((NP,), jnp.float32),
        ],
    )
    return f(col2d, w2d)


def _sc_prop(row1d, col2d, w2d, y):
    f = pl.kernel(
        _sc_prop_body,
        out_type=jax.ShapeDtypeStruct((NUM_CORES, NP), jnp.float32),
        mesh=_sc_mesh(),
        scratch_types=[
            pltpu.VMEM((NP,), jnp.float32),
            pltpu.VMEM((CHUNK,), jnp.int32),
            pltpu.VMEM((CHUNK // 128, 128), jnp.int32),
            pltpu.VMEM((CHUNK // 128, 128), jnp.float32),
            pltpu.VMEM((CHUNK // 128, 128), jnp.float32),
            pltpu.VMEM((NODE_SLICE,), jnp.float32),
            pltpu.VMEM_SHARED((NP,), jnp.float32),
        ],
    )
    return f(row1d, col2d, w2d, y)


_MLP_ROWS = 4000


def _mlp_body(a_ref, wm_ref, b_ref, o_ref):
    acc = jnp.dot(a_ref[...], wm_ref[...], preferred_element_type=jnp.float32)
    o_ref[...] = jax.nn.sigmoid(acc + b_ref[0, 0])


def _edge_mlp(edge_attr, W_edge, b_edge):
    a2d = edge_attr.reshape(E // 8, 128)
    wmat = jnp.kron(jnp.eye(8, dtype=jnp.float32), W_edge)
    b2d = b_edge.reshape(1, 1)
    grid = (E // 8) // _MLP_ROWS
    out8 = pl.pallas_call(
        _mlp_body,
        grid=(grid,),
        in_specs=[
            pl.BlockSpec((_MLP_ROWS, 128), lambda i: (i, 0)),
            pl.BlockSpec((128, 8), lambda i: (0, 0)),
            pl.BlockSpec(memory_space=pltpu.SMEM),
        ],
        out_specs=pl.BlockSpec((_MLP_ROWS, 8), lambda i: (i, 0)),
        out_shape=jax.ShapeDtypeStruct((E // 8, 8), jnp.float32),
    )(a2d, wmat, b2d)
    return out8.reshape(E)


def _combine0_body(d0_ref, d1_ref, m_ref, dis_ref, sc_ref, x_ref, y_ref):
    deg = 1.0 + d0_ref[...] + d1_ref[...]
    dis = lax.rsqrt(deg)
    x0 = jnp.maximum(m_ref[...], 0.0)
    dis_ref[...] = dis
    sc_ref[...] = dis * dis
    x_ref[...] = x0
    y_ref[...] = x0 * dis


def _combine0(d0, d1, maskp):
    shp = jax.ShapeDtypeStruct((NPR, 128), jnp.float32)
    return pl.pallas_call(
        _combine0_body,
        out_shape=(shp, shp, shp, shp),
    )(d0, d1, maskp)


def _combine_body(final, s0_ref, s1_ref, x_ref, h_ref, dis_ref, sc_ref,
                  a_ref, bias_ref, xn_ref, yn_ref, fill_ref):
    a = a_ref[0, 0]
    xn = (1.0 - a) * ((s0_ref[...] + s1_ref[...]) * dis_ref[...]
                      + sc_ref[...] * x_ref[...]) + a * h_ref[...]
    xn_ref[...] = xn
    yn_ref[...] = xn * dis_ref[...]
    if final:
        fill_ref[...] = jnp.tanh(xn - jnp.log1p(jnp.exp(bias_ref[0, 0])))
    else:
        fill_ref[...] = jnp.zeros_like(xn)


def _combine(s0, s1, x, h, dis, sc, a2d, bias2d, final):
    shp = jax.ShapeDtypeStruct((NPR, 128), jnp.float32)
    return pl.pallas_call(
        functools.partial(_combine_body, final),
        in_specs=[pl.BlockSpec((NPR, 128), lambda: (0, 0))] * 6
        + [pl.BlockSpec(memory_space=pltpu.SMEM)] * 2,
        out_shape=(shp, shp, shp),
    )(s0, s1, x, h, dis, sc, a2d, bias2d)


def kernel(edge_attr, mask, edge_index, W_edge, b_edge, alpha, bias):
    row = edge_index[0].astype(jnp.int32)
    col = edge_index[1].astype(jnp.int32)

    edge_weights = _edge_mlp(edge_attr, W_edge, b_edge)

    pad_e = EP - E
    rowp = jnp.concatenate([row, jnp.zeros((pad_e,), jnp.int32)])
    colp = jnp.concatenate([col, jnp.full((pad_e,), NP - 1, jnp.int32)])
    wp = jnp.concatenate([edge_weights, jnp.zeros((pad_e,), jnp.float32)])
    col2d = colp.reshape(EPR, 128)
    w2d = wp.reshape(EPR, 128)

    maskp = jnp.concatenate([mask[:, 0], jnp.zeros((NP - N,), jnp.float32)])
    maskp = maskp.reshape(NPR, 128)

    deg = _sc_deg(col2d, w2d)
    d0 = deg[0].reshape(NPR, 128)
    d1 = deg[1].reshape(NPR, 128)
    dis, sc, x, y = _combine0(d0, d1, maskp)
    h = x

    a2d = alpha.reshape(1, 1)
    bias2d = bias.reshape(1, 1)

    fill = None
    for k in range(K):
        s = _sc_prop(rowp, col2d, w2d, y.reshape(NP))
        s0 = s[0].reshape(NPR, 128)
        s1 = s[1].reshape(NPR, 128)
        x, y, fill = _combine(s0, s1, x, h, dis, sc, a2d, bias2d,
                              final=(k == K - 1))

    out = fill.reshape(NP)[:N].reshape(N, 1)
    return (out, edge_weights)


# speedup vs baseline: 73.6456x; 73.6456x over previous
import functools

import jax
import jax.numpy as jnp
from jax import lax
from jax.experimental import pallas as pl
from jax.experimental.pallas import tpu as pltpu
from jax.experimental.pallas import tpu_sc as plsc

N = 100000
E = 3200000
K = 5

NUM_CORES = 2
NUM_SUBCORES = 16
NUM_TILES = NUM_CORES * NUM_SUBCORES

NODE_SLICE = 6256
NP = NUM_SUBCORES * NODE_SLICE
NPR = NP // 128

CHUNK = 2048
TILE_CHUNKS = 49
TILE_EDGES = CHUNK * TILE_CHUNKS
EP = NUM_TILES * TILE_EDGES
EPR = EP // 128


def _sc_mesh():
    return plsc.VectorSubcoreMesh(
        core_axis_name="c", subcore_axis_name="s",
        num_cores=NUM_CORES, num_subcores=NUM_SUBCORES)


def _zero_spmem_slice(zbuf, agg_sh, tid):
    def zb(i, _):
        zbuf[pl.ds(i * 16, 16)] = jnp.zeros((16,), jnp.float32)
        return 0
    lax.fori_loop(0, NODE_SLICE // 16, zb, 0)
    pltpu.sync_copy(zbuf, agg_sh.at[pl.ds(tid * NODE_SLICE, NODE_SLICE)])


def _sc_deg_body(col_hbm, w_hbm, out_hbm, col_v, w_v, zbuf, agg_sh):
    cid = lax.axis_index("c")
    tid = lax.axis_index("s")
    wid = cid * NUM_SUBCORES + tid
    ebase = wid * TILE_EDGES
    _zero_spmem_slice(zbuf, agg_sh, tid)
    plsc.subcore_barrier()

    def chunk_body(c, _):
        eb = ebase + c * CHUNK
        pltpu.sync_copy(col_hbm.at[pl.ds(eb, CHUNK)], col_v)
        pltpu.sync_copy(w_hbm.at[pl.ds(eb, CHUNK)], w_v)
        pltpu.sync_copy(w_v, agg_sh.at[col_v], add=True)
        return 0
    lax.fori_loop(0, TILE_CHUNKS, chunk_body, 0)

    plsc.subcore_barrier()
    nb = tid * NODE_SLICE
    pltpu.sync_copy(agg_sh.at[pl.ds(nb, NODE_SLICE)], zbuf)
    pltpu.sync_copy(zbuf, out_hbm.at[pl.ds(cid * NP + nb, NODE_SLICE)])


def _sc_prop_body(row_hbm, col_hbm, w_hbm, y_hbm, out_hbm,
                  y_v, row_v, col_v, w_v, msg_v, zbuf, agg_sh):
    cid = lax.axis_index("c")
    tid = lax.axis_index("s")
    wid = cid * NUM_SUBCORES + tid
    ebase = wid * TILE_EDGES
    _zero_spmem_slice(zbuf, agg_sh, tid)
    pltpu.sync_copy(y_hbm, y_v)
    plsc.subcore_barrier()

    def chunk_body(c, _):
        eb = ebase + c * CHUNK
        pltpu.sync_copy(row_hbm.at[pl.ds(eb, CHUNK)], row_v)
        pltpu.sync_copy(col_hbm.at[pl.ds(eb, CHUNK)], col_v)
        pltpu.sync_copy(w_hbm.at[pl.ds(eb, CHUNK)], w_v)

        def g(i, _):
            idx = row_v[pl.ds(i * 16, 16)]
            xv = plsc.load_gather(y_v, [idx])
            msg_v[pl.ds(i * 16, 16)] = xv * w_v[pl.ds(i * 16, 16)]
            return 0
        lax.fori_loop(0, CHUNK // 16, g, 0)
        pltpu.sync_copy(msg_v, agg_sh.at[col_v], add=True)
        return 0
    lax.fori_loop(0, TILE_CHUNKS, chunk_body, 0)

    plsc.subcore_barrier()
    nb = tid * NODE_SLICE
    pltpu.sync_copy(agg_sh.at[pl.ds(nb, NODE_SLICE)], zbuf)
    pltpu.sync_copy(zbuf, out_hbm.at[pl.ds(cid * NP + nb, NODE_SLICE)])


def _sc_deg(col2d, w2d):
    f = pl.kernel(
        _sc_deg_body,
        out_type=jax.ShapeDtypeStruct((NUM_CORES * NP,), jnp.float32),
        mesh=_sc_mesh(),
        compiler_params=pltpu.CompilerParams(needs_layout_passes=False),
        scratch_types=[
            pltpu.VMEM((CHUNK,), jnp.int32),
            pltpu.VMEM((CHUNK,), jnp.float32),
            pltpu.VMEM((NODE_SLICE,), jnp.float32),
            pltpu.VMEM_SHARED((NP,), jnp.float32),
        ],
    )
    return f(col2d, w2d)


def _sc_prop(row1d, col2d, w2d, y):
    f = pl.kernel(
        _sc_prop_body,
        out_type=jax.ShapeDtypeStruct((NUM_CORES * NP,), jnp.float32),
        mesh=_sc_mesh(),
        compiler_params=pltpu.CompilerParams(needs_layout_passes=False),
        scratch_types=[
            pltpu.VMEM((NP,), jnp.float32),
            pltpu.VMEM((CHUNK,), jnp.int32),
            pltpu.VMEM((CHUNK,), jnp.int32),
            pltpu.VMEM((CHUNK,), jnp.float32),
            pltpu.VMEM((CHUNK,), jnp.float32),
            pltpu.VMEM((NODE_SLICE,), jnp.float32),
            pltpu.VMEM_SHARED((NP,), jnp.float32),
        ],
    )
    return f(row1d, col2d, w2d, y)


_MLP_ROWS = 4000


def _mlp_body(a_ref, wm_ref, b_ref, o_ref):
    acc = jnp.dot(a_ref[...], wm_ref[...], preferred_element_type=jnp.float32)
    o_ref[...] = jax.nn.sigmoid(acc + b_ref[0, 0])


def _edge_mlp(edge_attr, W_edge, b_edge):
    a2d = edge_attr.reshape(E // 8, 128)
    wmat = jnp.kron(jnp.eye(8, dtype=jnp.float32), W_edge)
    b2d = b_edge.reshape(1, 1)
    grid = (E // 8) // _MLP_ROWS
    out8 = pl.pallas_call(
        _mlp_body,
        grid=(grid,),
        in_specs=[
            pl.BlockSpec((_MLP_ROWS, 128), lambda i: (i, 0)),
            pl.BlockSpec((128, 8), lambda i: (0, 0)),
            pl.BlockSpec(memory_space=pltpu.SMEM),
        ],
        out_specs=pl.BlockSpec((_MLP_ROWS, 8), lambda i: (i, 0)),
        out_shape=jax.ShapeDtypeStruct((E // 8, 8), jnp.float32),
    )(a2d, wmat, b2d)
    return out8.reshape(E)


def _combine0_body(d0_ref, d1_ref, m_ref, dis_ref, sc_ref, x_ref, y_ref):
    deg = 1.0 + d0_ref[...] + d1_ref[...]
    dis = lax.rsqrt(deg)
    x0 = jnp.maximum(m_ref[...], 0.0)
    dis_ref[...] = dis
    sc_ref[...] = dis * dis
    x_ref[...] = x0
    y_ref[...] = x0 * dis


def _combine0(d0, d1, maskp):
    shp = jax.ShapeDtypeStruct((NPR, 128), jnp.float32)
    return pl.pallas_call(
        _combine0_body,
        out_shape=(shp, shp, shp, shp),
    )(d0, d1, maskp)


def _combine_body(final, s0_ref, s1_ref, x_ref, h_ref, dis_ref, sc_ref,
                  a_ref, bias_ref, xn_ref, yn_ref, fill_ref):
    a = a_ref[0, 0]
    xn = (1.0 - a) * ((s0_ref[...] + s1_ref[...]) * dis_ref[...]
                      + sc_ref[...] * x_ref[...]) + a * h_ref[...]
    xn_ref[...] = xn
    yn_ref[...] = xn * dis_ref[...]
    if final:
        fill_ref[...] = jnp.tanh(xn - jnp.log1p(jnp.exp(bias_ref[0, 0])))
    else:
        fill_ref[...] = jnp.zeros_like(xn)


def _combine(s0, s1, x, h, dis, sc, a2d, bias2d, final):
    shp = jax.ShapeDtypeStruct((NPR, 128), jnp.float32)
    return pl.pallas_call(
        functools.partial(_combine_body, final),
        in_specs=[pl.BlockSpec((NPR, 128), lambda: (0, 0))] * 6
        + [pl.BlockSpec(memory_space=pltpu.SMEM)] * 2,
        out_shape=(shp, shp, shp),
    )(s0, s1, x, h, dis, sc, a2d, bias2d)


def kernel(edge_attr, mask, edge_index, W_edge, b_edge, alpha, bias):
    row = edge_index[0].astype(jnp.int32)
    col = edge_index[1].astype(jnp.int32)

    edge_weights = _edge_mlp(edge_attr, W_edge, b_edge)

    pad_e = EP - E
    rowp = jnp.concatenate([row, jnp.zeros((pad_e,), jnp.int32)])
    colp = jnp.concatenate([col, jnp.full((pad_e,), NP - 1, jnp.int32)])
    wp = jnp.concatenate([edge_weights, jnp.zeros((pad_e,), jnp.float32)])
    col2d = colp
    w2d = wp

    maskp = jnp.concatenate([mask[:, 0], jnp.zeros((NP - N,), jnp.float32)])
    maskp = maskp.reshape(NPR, 128)

    deg = _sc_deg(col2d, w2d).reshape(NUM_CORES, NP)
    d0 = deg[0].reshape(NPR, 128)
    d1 = deg[1].reshape(NPR, 128)
    dis, sc, x, y = _combine0(d0, d1, maskp)
    h = x

    a2d = alpha.reshape(1, 1)
    bias2d = bias.reshape(1, 1)

    fill = None
    for k in range(K):
        s = _sc_prop(rowp, col2d, w2d, y.reshape(NP)).reshape(NUM_CORES, NP)
        s0 = s[0].reshape(NPR, 128)
        s1 = s[1].reshape(NPR, 128)
        x, y, fill = _combine(s0, s1, x, h, dis, sc, a2d, bias2d,
                              final=(k == K - 1))

    out = fill.reshape(NP)[:N].reshape(N, 1)
    return (out, edge_weights)


# speedup vs baseline: 101.7439x; 1.3815x over previous
import functools

import jax
import jax.numpy as jnp
from jax import lax
from jax.experimental import pallas as pl
from jax.experimental.pallas import tpu as pltpu
from jax.experimental.pallas import tpu_sc as plsc

N = 100000
E = 3200000
K = 5

NUM_CORES = 2
NUM_SUBCORES = 16
NUM_TILES = NUM_CORES * NUM_SUBCORES

NODE_SLICE = 6256
NP = NUM_SUBCORES * NODE_SLICE
NPR = NP // 128

CHUNK = 2000
TILE_CHUNKS = 50
TILE_EDGES = CHUNK * TILE_CHUNKS
PEEL = 6


def _sc_mesh():
    return plsc.VectorSubcoreMesh(
        core_axis_name="c", subcore_axis_name="s",
        num_cores=NUM_CORES, num_subcores=NUM_SUBCORES)


def _zero_spmem_slice(zbuf, agg_sh, tid):
    def zb(i, _):
        zbuf[pl.ds(i * 16, 16)] = jnp.zeros((16,), jnp.float32)
        return 0
    lax.fori_loop(0, NODE_SLICE // 16, zb, 0)
    pltpu.sync_copy(zbuf, agg_sh.at[pl.ds(tid * NODE_SLICE, NODE_SLICE)])


def _sc_prop_body(eflat_hbm, w_hbm, y_hbm, out_hbm,
                  y_v, row_v0, row_v1, w_v0, w_v1,
                  col_v0, col_v1, col_v2, msg_v0, msg_v1, msg_v2,
                  agg_sh,
                  sem_y, sem_rw0, sem_rw1, sem_c0, sem_c1, sem_c2,
                  sem_s0, sem_s1, sem_s2):
    row_v = (row_v0, row_v1)
    w_v = (w_v0, w_v1)
    col_v = (col_v0, col_v1, col_v2)
    msg_v = (msg_v0, msg_v1, msg_v2)
    sem_rw = (sem_rw0, sem_rw1)
    sem_c = (sem_c0, sem_c1, sem_c2)
    sem_s = (sem_s0, sem_s1, sem_s2)

    cid = lax.axis_index("c")
    tid = lax.axis_index("s")
    wid = cid * NUM_SUBCORES + tid
    ebase = wid * TILE_EDGES

    def start_loads(c, b2, b3):
        eb = ebase + c * CHUNK
        pltpu.async_copy(eflat_hbm.at[pl.ds(eb, CHUNK)], row_v[b2], sem_rw[b2])
        pltpu.async_copy(w_hbm.at[pl.ds(eb, CHUNK)], w_v[b2], sem_rw[b2])
        pltpu.async_copy(eflat_hbm.at[pl.ds(E + eb, CHUNK)], col_v[b3],
                         sem_c[b3])

    def wait_loads(c, b2, b3):
        eb = ebase + c * CHUNK
        pltpu.make_async_copy(eflat_hbm.at[pl.ds(eb, CHUNK)], row_v[b2],
                              sem_rw[b2]).wait()
        pltpu.make_async_copy(w_hbm.at[pl.ds(eb, CHUNK)], w_v[b2],
                              sem_rw[b2]).wait()
        pltpu.make_async_copy(eflat_hbm.at[pl.ds(E + eb, CHUNK)], col_v[b3],
                              sem_c[b3]).wait()

    def wait_scatter(b3):
        pltpu.make_async_copy(msg_v[b3], agg_sh.at[col_v[b3]],
                              sem_s[b3]).wait()

    def do_chunk(c, b2, b3, start_next, wait_prev):
        nb2 = (b2 + 1) % 2
        nb3 = (b3 + 1) % 3
        if wait_prev:
            wait_scatter(nb3)
        if start_next:
            start_loads(c + 1, nb2, nb3)
        wait_loads(c, b2, b3)

        def g(i, _):
            idx = row_v[b2][pl.ds(i * 16, 16)]
            xv = plsc.load_gather(y_v, [idx])
            msg_v[b3][pl.ds(i * 16, 16)] = xv * w_v[b2][pl.ds(i * 16, 16)]
            return 0
        lax.fori_loop(0, CHUNK // 16, g, 0)
        pltpu.async_copy(msg_v[b3], agg_sh.at[col_v[b3]], sem_s[b3], add=True)

    _zero_spmem_slice(y_v.at[pl.ds(0, NODE_SLICE)], agg_sh, tid)
    pltpu.async_copy(y_hbm, y_v, sem_y)
    start_loads(0, 0, 0)
    pltpu.make_async_copy(y_hbm, y_v, sem_y).wait()
    plsc.subcore_barrier()

    for c in range(PEEL):
        do_chunk(c, c % 2, c % 3, start_next=True, wait_prev=(c >= 2))

    def outer(o, _):
        base = o * 6
        for j in range(6):
            do_chunk(base + j, j % 2, j % 3, start_next=True, wait_prev=True)
        return 0
    lax.fori_loop(1, (TILE_CHUNKS - 2) // 6, outer, 0)

    do_chunk(TILE_CHUNKS - 2, 0, 0, start_next=True, wait_prev=True)
    do_chunk(TILE_CHUNKS - 1, 1, 1, start_next=False, wait_prev=False)
    for b3 in range(3):
        wait_scatter(b3)

    plsc.subcore_barrier()
    nb = tid * NODE_SLICE
    bounce = y_v.at[pl.ds(0, NODE_SLICE)]
    pltpu.sync_copy(agg_sh.at[pl.ds(nb, NODE_SLICE)], bounce)
    pltpu.sync_copy(bounce, out_hbm.at[pl.ds(cid * NP + nb, NODE_SLICE)])


def _sc_deg_body(eflat_hbm, w_hbm, out_hbm,
                 col_v0, col_v1, col_v2, w_v0, w_v1, w_v2,
                 zbuf, agg_sh,
                 sem_c0, sem_c1, sem_c2, sem_s0, sem_s1, sem_s2):
    col_v = (col_v0, col_v1, col_v2)
    w_v = (w_v0, w_v1, w_v2)
    sem_c = (sem_c0, sem_c1, sem_c2)
    sem_s = (sem_s0, sem_s1, sem_s2)

    cid = lax.axis_index("c")
    tid = lax.axis_index("s")
    wid = cid * NUM_SUBCORES + tid
    ebase = wid * TILE_EDGES

    def start_loads(c, b3):
        eb = ebase + c * CHUNK
        pltpu.async_copy(w_hbm.at[pl.ds(eb, CHUNK)], w_v[b3], sem_c[b3])
        pltpu.async_copy(eflat_hbm.at[pl.ds(E + eb, CHUNK)], col_v[b3],
                         sem_c[b3])

    def wait_loads(c, b3):
        eb = ebase + c * CHUNK
        pltpu.make_async_copy(w_hbm.at[pl.ds(eb, CHUNK)], w_v[b3],
                              sem_c[b3]).wait()
        pltpu.make_async_copy(eflat_hbm.at[pl.ds(E + eb, CHUNK)], col_v[b3],
                              sem_c[b3]).wait()

    def wait_scatter(b3):
        pltpu.make_async_copy(w_v[b3], agg_sh.at[col_v[b3]], sem_s[b3]).wait()

    def do_chunk(c, b3, start_next, wait_prev):
        nb3 = (b3 + 1) % 3
        if wait_prev:
            wait_scatter(nb3)
        if start_next:
            start_loads(c + 1, nb3)
        wait_loads(c, b3)
        pltpu.async_copy(w_v[b3], agg_sh.at[col_v[b3]], sem_s[b3], add=True)

    _zero_spmem_slice(zbuf, agg_sh, tid)
    start_loads(0, 0)
    plsc.subcore_barrier()

    for c in range(PEEL):
        do_chunk(c, c % 3, start_next=True, wait_prev=(c >= 2))

    def outer(o, _):
        base = o * 6
        for j in range(6):
            do_chunk(base + j, j % 3, start_next=True, wait_prev=True)
        return 0
    lax.fori_loop(1, (TILE_CHUNKS - 2) // 6, outer, 0)

    do_chunk(TILE_CHUNKS - 2, 0, start_next=True, wait_prev=True)
    do_chunk(TILE_CHUNKS - 1, 1, start_next=False, wait_prev=False)
    for b3 in range(3):
        wait_scatter(b3)

    plsc.subcore_barrier()
    nb = tid * NODE_SLICE
    pltpu.sync_copy(agg_sh.at[pl.ds(nb, NODE_SLICE)], zbuf)
    pltpu.sync_copy(zbuf, out_hbm.at[pl.ds(cid * NP + nb, NODE_SLICE)])


def _sc_deg(eflat, w):
    f = pl.kernel(
        _sc_deg_body,
        out_type=jax.ShapeDtypeStruct((NUM_CORES * NP,), jnp.float32),
        mesh=_sc_mesh(),
        compiler_params=pltpu.CompilerParams(needs_layout_passes=False),
        scratch_types=[
            pltpu.VMEM((CHUNK,), jnp.int32),
            pltpu.VMEM((CHUNK,), jnp.int32),
            pltpu.VMEM((CHUNK,), jnp.int32),
            pltpu.VMEM((CHUNK,), jnp.float32),
            pltpu.VMEM((CHUNK,), jnp.float32),
            pltpu.VMEM((CHUNK,), jnp.float32),
            pltpu.VMEM((NODE_SLICE,), jnp.float32),
            pltpu.VMEM_SHARED((NP,), jnp.float32),
        ] + [pltpu.SemaphoreType.DMA] * 6,
    )
    return f(eflat, w)


def _sc_prop(eflat, w, y):
    f = pl.kernel(
        _sc_prop_body,
        out_type=jax.ShapeDtypeStruct((NUM_CORES * NP,), jnp.float32),
        mesh=_sc_mesh(),
        compiler_params=pltpu.CompilerParams(needs_layout_passes=False),
        scratch_types=[
            pltpu.VMEM((NP,), jnp.float32),
            pltpu.VMEM((CHUNK,), jnp.int32),
            pltpu.VMEM((CHUNK,), jnp.int32),
            pltpu.VMEM((CHUNK,), jnp.float32),
            pltpu.VMEM((CHUNK,), jnp.float32),
            pltpu.VMEM((CHUNK,), jnp.int32),
            pltpu.VMEM((CHUNK,), jnp.int32),
            pltpu.VMEM((CHUNK,), jnp.int32),
            pltpu.VMEM((CHUNK,), jnp.float32),
            pltpu.VMEM((CHUNK,), jnp.float32),
            pltpu.VMEM((CHUNK,), jnp.float32),
            pltpu.VMEM_SHARED((NP,), jnp.float32),
        ] + [pltpu.SemaphoreType.DMA] * 9,
    )
    return f(eflat, w, y)


_MLP_ROWS = 4000


def _mlp_body(a_ref, wm_ref, b_ref, o_ref):
    acc = jnp.dot(a_ref[...], wm_ref[...], preferred_element_type=jnp.float32)
    o_ref[...] = jax.nn.sigmoid(acc + b_ref[0, 0])


def _edge_mlp(edge_attr, W_edge, b_edge):
    a2d = edge_attr.reshape(E // 8, 128)
    wmat = jnp.kron(jnp.eye(8, dtype=jnp.float32), W_edge)
    b2d = b_edge.reshape(1, 1)
    grid = (E // 8) // _MLP_ROWS
    out8 = pl.pallas_call(
        _mlp_body,
        grid=(grid,),
        in_specs=[
            pl.BlockSpec((_MLP_ROWS, 128), lambda i: (i, 0)),
            pl.BlockSpec((128, 8), lambda i: (0, 0)),
            pl.BlockSpec(memory_space=pltpu.SMEM),
        ],
        out_specs=pl.BlockSpec((_MLP_ROWS, 8), lambda i: (i, 0)),
        out_shape=jax.ShapeDtypeStruct((E // 8, 8), jnp.float32),
    )(a2d, wmat, b2d)
    return out8.reshape(E)


def _combine0_body(d0_ref, d1_ref, m_ref, dis_ref, sc_ref, x_ref, y_ref):
    deg = 1.0 + d0_ref[...] + d1_ref[...]
    dis = lax.rsqrt(deg)
    x0 = jnp.maximum(m_ref[...], 0.0)
    dis_ref[...] = dis
    sc_ref[...] = dis * dis
    x_ref[...] = x0
    y_ref[...] = x0 * dis


def _combine0(d0, d1, maskp):
    shp = jax.ShapeDtypeStruct((NPR, 128), jnp.float32)
    return pl.pallas_call(
        _combine0_body,
        out_shape=(shp, shp, shp, shp),
    )(d0, d1, maskp)


def _combine_body(final, s0_ref, s1_ref, x_ref, h_ref, dis_ref, sc_ref,
                  a_ref, bias_ref, xn_ref, yn_ref, fill_ref):
    a = a_ref[0, 0]
    xn = (1.0 - a) * ((s0_ref[...] + s1_ref[...]) * dis_ref[...]
                      + sc_ref[...] * x_ref[...]) + a * h_ref[...]
    xn_ref[...] = xn
    yn_ref[...] = xn * dis_ref[...]
    if final:
        fill_ref[...] = jnp.tanh(xn - jnp.log1p(jnp.exp(bias_ref[0, 0])))
    else:
        fill_ref[...] = jnp.zeros_like(xn)


def _combine(s0, s1, x, h, dis, sc, a2d, bias2d, final):
    shp = jax.ShapeDtypeStruct((NPR, 128), jnp.float32)
    return pl.pallas_call(
        functools.partial(_combine_body, final),
        in_specs=[pl.BlockSpec((NPR, 128), lambda: (0, 0))] * 6
        + [pl.BlockSpec(memory_space=pltpu.SMEM)] * 2,
        out_shape=(shp, shp, shp),
    )(s0, s1, x, h, dis, sc, a2d, bias2d)


def kernel(edge_attr, mask, edge_index, W_edge, b_edge, alpha, bias):
    eflat = edge_index.astype(jnp.int32).reshape(2 * E)

    edge_weights = _edge_mlp(edge_attr, W_edge, b_edge)

    maskp = jnp.concatenate([mask[:, 0], jnp.zeros((NP - N,), jnp.float32)])
    maskp = maskp.reshape(NPR, 128)

    deg = _sc_deg(eflat, edge_weights).reshape(NUM_CORES, NP)
    d0 = deg[0].reshape(NPR, 128)
    d1 = deg[1].reshape(NPR, 128)
    dis, sc, x, y = _combine0(d0, d1, maskp)
    h = x

    a2d = alpha.reshape(1, 1)
    bias2d = bias.reshape(1, 1)

    fill = None
    for k in range(K):
        s = _sc_prop(eflat, edge_weights, y.reshape(NP)).reshape(NUM_CORES, NP)
        s0 = s[0].reshape(NPR, 128)
        s1 = s[1].reshape(NPR, 128)
        x, y, fill = _combine(s0, s1, x, h, dis, sc, a2d, bias2d,
                              final=(k == K - 1))

    out = fill.reshape(NP)[:N].reshape(N, 1)
    return (out, edge_weights)
